# baseline (speedup 1.0000x reference)
import jax
import jax.numpy as jnp
from jax import lax
from jax.experimental import pallas as pl
from jax.experimental.pallas import tpu as pltpu
from jax.experimental.pallas import tpu_sc as plsc

N = 10000
E = 160000
CH0 = 256
H = 512
G = 64
DESC = 128

CK = 64
NCK = H // CK
NT = 16
NC = 2
NPC = NCK // NC
EPT = E // NT
B = 128
D = 8
NGRP = (EPT + B * D - 1) // (B * D)
NB = NGRP * D
EPP = NB * B
RPT = 632
RPT_LAST = N - (NT - 1) * RPT
ACC_ROWS = NT * RPT

_f32 = jnp.float32


def _copy_rows(s, src_ref, dst_ref):
  r0 = pl.multiple_of(s * RPT, 8)

  @pl.when(s < NT - 1)
  def _():
    pltpu.sync_copy(src_ref.at[pl.ds(r0, RPT)], dst_ref.at[pl.ds(r0, RPT)])

  @pl.when(s == NT - 1)
  def _():
    base = (NT - 1) * RPT
    pltpu.sync_copy(src_ref.at[pl.ds(base, RPT_LAST)],
                    dst_ref.at[pl.ds(base, RPT_LAST)])


def _sc_mesh():
  return plsc.VectorSubcoreMesh(
      core_axis_name="c", subcore_axis_name="s",
      num_cores=NC, num_subcores=NT)


def _agg_body(*refs):
  ys = refs[:NCK]
  srcp, dstp = refs[NCK:NCK + 2]
  os_ = refs[NCK + 2:2 * NCK + 2]
  src_v, dst_v, bufs, gsems, ssems, acc = refs[2 * NCK + 2:]
  c = lax.axis_index("c")
  s = lax.axis_index("s")
  pltpu.sync_copy(srcp.at[s], src_v)
  pltpu.sync_copy(dstp.at[s], dst_v)

  def process(y_ref, o_ref):
    _copy_rows(s, y_ref, acc)
    plsc.subcore_barrier()

    def body(g, carry):
      base = g * D
      descs = []
      for k in range(D):
        @pl.when(g >= 1)
        def _(k=k):
          pltpu.make_async_copy(
              y_ref.at[pl.ds(0, B)], bufs.at[k], ssems.at[k]).wait()
        descs.append(pltpu.async_copy(
            y_ref.at[src_v.at[base + k]], bufs.at[k], gsems.at[k]))
      for k in range(D):
        descs[k].wait()
        pltpu.async_copy(bufs.at[k], acc.at[dst_v.at[base + k]],
                         ssems.at[k], add=True)
      return carry

    lax.fori_loop(0, NGRP, body, 0)
    for k in range(D):
      pltpu.make_async_copy(
          y_ref.at[pl.ds(0, B)], bufs.at[k], ssems.at[k]).wait()
    plsc.subcore_barrier()
    _copy_rows(s, acc, o_ref)
    plsc.subcore_barrier()

  @pl.when(c == 0)
  def _():
    for k in range(NPC):
      process(ys[k], os_[k])

  @pl.when(c == 1)
  def _():
    for k in range(NPC):
      process(ys[NPC + k], os_[NPC + k])


def _agg_call(y_chunks, srcp, dstp):
  fn = pl.kernel(
      _agg_body,
      out_type=[jax.ShapeDtypeStruct((N, CK), _f32)] * NCK,
      mesh=_sc_mesh(),
      scratch_types=[
          pltpu.VMEM((NB, B), jnp.int32),
          pltpu.VMEM((NB, B), jnp.int32),
          pltpu.VMEM((D, B, CK), _f32),
          pltpu.SemaphoreType.DMA((D,)),
          pltpu.SemaphoreType.DMA((D,)),
          pltpu.VMEM_SHARED((ACC_ROWS, CK), _f32),
      ],
      compiler_params=pltpu.CompilerParams(use_tc_tiling_on_sc=False),
  )
  return fn(*y_chunks, srcp, dstp)


def _deg_body(dstp, ones_h, zeros_h, degf, dst_v, buf, acc):
  c = lax.axis_index("c")
  s = lax.axis_index("s")

  @pl.when(c == 0)
  def _():
    pltpu.sync_copy(dstp.at[s], dst_v)
    pltpu.sync_copy(ones_h, buf)
    z0 = pl.multiple_of(s * RPT, 8)
    pltpu.sync_copy(zeros_h.at[pl.ds(z0, RPT)], acc.at[pl.ds(z0, RPT)])
    plsc.subcore_barrier()

    def body(j, carry):
      pltpu.sync_copy(buf, acc.at[dst_v.at[j]], add=True)
      return carry

    lax.fori_loop(0, NB, body, 0)
    plsc.subcore_barrier()
    _copy_rows(s, acc, degf)


def _deg_call(dstp, ones_h, zeros_h):
  fn = pl.kernel(
      _deg_body,
      out_type=jax.ShapeDtypeStruct((N, CK), _f32),
      mesh=_sc_mesh(),
      scratch_types=[
          pltpu.VMEM((NB, B), jnp.int32),
          pltpu.VMEM((B, CK), _f32),
          pltpu.VMEM_SHARED((ACC_ROWS, CK), _f32),
      ],
      compiler_params=pltpu.CompilerParams(use_tc_tiling_on_sc=False),
  )
  return fn(dstp, ones_h, zeros_h)


R0 = 1000


def _tc0_body(x_ref, w_ref, deg_ref, *y_refs):
  dinv = lax.rsqrt(deg_ref[:, 0:1] + 1.0)
  y = jnp.dot(x_ref[...] * dinv, w_ref[...], preferred_element_type=_f32)
  for k, yr in enumerate(y_refs):
    yr[...] = y[:, k * CK:(k + 1) * CK]


def _tc0(x, W, degf):
  return pl.pallas_call(
      _tc0_body,
      grid=(N // R0,),
      in_specs=[
          pl.BlockSpec((R0, CH0), lambda i: (i, 0)),
          pl.BlockSpec((CH0, H), lambda i: (0, 0)),
          pl.BlockSpec((R0, CK), lambda i: (i, 0)),
      ],
      out_specs=[pl.BlockSpec((R0, CK), lambda i: (i, 0))] * NCK,
      out_shape=[jax.ShapeDtypeStruct((N, CK), _f32)] * NCK,
  )(x, W, degf)


def _tcmid_body(*refs):
  a_refs = refs[:NCK]
  w_ref, deg_ref, b_ref = refs[NCK:NCK + 3]
  y_refs = refs[NCK + 3:]
  dinv = lax.rsqrt(deg_ref[:, 0:1] + 1.0)
  h = jnp.concatenate([a[...] for a in a_refs], axis=1)
  h = jnp.maximum(h * dinv + b_ref[...], 0.0)
  y = jnp.dot(h * dinv, w_ref[...], preferred_element_type=_f32)
  for k, yr in enumerate(y_refs):
    yr[...] = y[:, k * CK:(k + 1) * CK]


def _tcmid(acc, W, b, degf):
  return pl.pallas_call(
      _tcmid_body,
      grid=(N // R0,),
      in_specs=[pl.BlockSpec((R0, CK), lambda i: (i, 0))] * NCK + [
          pl.BlockSpec((H, H), lambda i: (0, 0)),
          pl.BlockSpec((R0, CK), lambda i: (i, 0)),
          pl.BlockSpec((1, H), lambda i: (0, 0)),
      ],
      out_specs=[pl.BlockSpec((R0, CK), lambda i: (i, 0))] * NCK,
      out_shape=[jax.ShapeDtypeStruct((N, CK), _f32)] * NCK,
  )(*acc, W, degf, b.reshape(1, H))


RF = 400
NGF = N // RF


def _fin_body(*refs):
  a_refs = refs[:NCK]
  (deg_ref, b_ref, batch_ref, desc_ref, wd_ref, bd_ref, wl_ref, bl_ref,
   out_ref, sums, counts) = refs[NCK:]
  i = pl.program_id(0)

  @pl.when(i == 0)
  def _():
    sums[...] = jnp.zeros_like(sums)
    counts[...] = jnp.zeros_like(counts)

  dinv = lax.rsqrt(deg_ref[:, 0:1] + 1.0)
  h = jnp.concatenate([a[...] for a in a_refs], axis=1)
  h = jnp.maximum(h * dinv + b_ref[...], 0.0)
  gids = lax.broadcasted_iota(jnp.int32, (RF, G), 1)
  P = (batch_ref[...] == gids).astype(_f32)
  sums[...] += lax.dot_general(P, h, (((0,), (0,)), ((), ())),
                               preferred_element_type=_f32)
  counts[...] += lax.dot_general(P, jnp.ones((RF, 1), _f32),
                                 (((0,), (0,)), ((), ())),
                                 preferred_element_type=_f32)

  @pl.when(i == NGF - 1)
  def _():
    gm = sums[...] / jnp.maximum(counts[...], 1.0)
    de = jnp.maximum(
        jnp.dot(desc_ref[...], wd_ref[...], preferred_element_type=_f32)
        + bd_ref[...], 0.0)
    z = jnp.concatenate([gm, de], axis=1)
    logit = jnp.dot(z, wl_ref[...], preferred_element_type=_f32) + bl_ref[...]
    out_ref[...] = jax.nn.sigmoid(logit)


def _tcfinal(acc, b, degf, batch, descriptors, Wd, bd, Wlin, blin):
  return pl.pallas_call(
      _fin_body,
      grid=(NGF,),
      in_specs=[pl.BlockSpec((RF, CK), lambda i: (i, 0))] * NCK + [
          pl.BlockSpec((RF, CK), lambda i: (i, 0)),
          pl.BlockSpec((1, H), lambda i: (0, 0)),
          pl.BlockSpec((RF, 1), lambda i: (i, 0)),
          pl.BlockSpec((G, DESC), lambda i: (0, 0)),
          pl.BlockSpec((DESC, H), lambda i: (0, 0)),
          pl.BlockSpec((1, H), lambda i: (0, 0)),
          pl.BlockSpec((2 * H, 1), lambda i: (0, 0)),
          pl.BlockSpec((1, 1), lambda i: (0, 0)),
      ],
      out_specs=pl.BlockSpec((G, 1), lambda i: (0, 0)),
      out_shape=jax.ShapeDtypeStruct((G, 1), _f32),
      scratch_shapes=[
          pltpu.VMEM((G, H), _f32),
          pltpu.VMEM((G, 1), _f32),
      ],
  )(*acc, degf, b.reshape(1, H), batch.reshape(N, 1), descriptors,
    Wd, bd.reshape(1, H), Wlin, blin.reshape(1, 1))


def kernel(x, edge_index, batch, descriptors,
           W0, b0, W1, b1, W2, b2, W3, b3, Wd, bd, Wlin, blin):
  src = edge_index[0].reshape(NT, EPT)
  dst = edge_index[1].reshape(NT, EPT)
  pad = EPP - EPT
  srcp = jnp.pad(src, ((0, 0), (0, pad)), constant_values=0).reshape(NT, NB, B)
  dstp = jnp.pad(dst, ((0, 0), (0, pad)), constant_values=N).reshape(NT, NB, B)
  ones_h = jnp.ones((B, CK), _f32)
  zeros_h = jnp.zeros((ACC_ROWS, CK), _f32)

  degf = _deg_call(dstp, ones_h, zeros_h)
  y = _tc0(x, W0, degf)
  bs = [b0, b1, b2, b3]
  Ws = [W1, W2, W3]
  for l in range(3):
    acc = _agg_call(y, srcp, dstp)
    y = _tcmid(acc, Ws[l], bs[l], degf)
  acc = _agg_call(y, srcp, dstp)
  out = _tcfinal(acc, bs[3], degf, batch, descriptors, Wd, bd, Wlin, blin)
  return out.reshape(-1)


# speedup vs baseline: 1.1808x; 1.1808x over previous
import jax
import jax.numpy as jnp
from jax import lax
from jax.experimental import pallas as pl
from jax.experimental.pallas import tpu as pltpu
from jax.experimental.pallas import tpu_sc as plsc

N = 10000
E = 160000
CH0 = 256
H = 512
G = 64
DESC = 128

CK = 64
NCK = H // CK
NT = 16
NC = 2
NPC = NCK // NC
EPT = E // NT
B = 128
D = 3
NGRP = (EPT + B * D - 1) // (B * D)
NB = NGRP * D
EPP = NB * B
RPT = 632
RPT_LAST = N - (NT - 1) * RPT
ACC_ROWS = NT * RPT

_f32 = jnp.float32


def _copy_rows(s, src_ref, dst_ref):
  r0 = pl.multiple_of(s * RPT, 8)

  @pl.when(s < NT - 1)
  def _():
    pltpu.sync_copy(src_ref.at[pl.ds(r0, RPT)], dst_ref.at[pl.ds(r0, RPT)])

  @pl.when(s == NT - 1)
  def _():
    base = (NT - 1) * RPT
    pltpu.sync_copy(src_ref.at[pl.ds(base, RPT_LAST)],
                    dst_ref.at[pl.ds(base, RPT_LAST)])


def _sc_mesh():
  return plsc.VectorSubcoreMesh(
      core_axis_name="c", subcore_axis_name="s",
      num_cores=NC, num_subcores=NT)


def _agg_body(*refs):
  ys = refs[:NCK]
  srcp, dstp = refs[NCK:NCK + 2]
  os_ = refs[NCK + 2:2 * NCK + 2]
  src_v, dst_v, bufs, gsems, ssems, acc, ysh = refs[2 * NCK + 2:]
  c = lax.axis_index("c")
  s = lax.axis_index("s")
  pltpu.sync_copy(srcp.at[s], src_v)
  pltpu.sync_copy(dstp.at[s], dst_v)

  def process(y_ref, o_ref):
    _copy_rows(s, y_ref, ysh)
    _copy_rows(s, y_ref, acc)
    plsc.subcore_barrier()

    def body(g, carry):
      base = g * D
      descs = []
      for k in range(D):
        @pl.when(g >= 1)
        def _(k=k):
          pltpu.make_async_copy(
              y_ref.at[pl.ds(0, B)], bufs.at[k], ssems.at[k]).wait()
        descs.append(pltpu.async_copy(
            ysh.at[src_v.at[base + k]], bufs.at[k], gsems.at[k]))
      for k in range(D):
        descs[k].wait()
        pltpu.async_copy(bufs.at[k], acc.at[dst_v.at[base + k]],
                         ssems.at[k], add=True)
      return carry

    lax.fori_loop(0, NGRP, body, 0)
    for k in range(D):
      pltpu.make_async_copy(
          y_ref.at[pl.ds(0, B)], bufs.at[k], ssems.at[k]).wait()
    plsc.subcore_barrier()
    _copy_rows(s, acc, o_ref)
    plsc.subcore_barrier()

  @pl.when(c == 0)
  def _():
    for k in range(NPC):
      process(ys[k], os_[k])

  @pl.when(c == 1)
  def _():
    for k in range(NPC):
      process(ys[NPC + k], os_[NPC + k])


def _agg_call(y_chunks, srcp, dstp):
  fn = pl.kernel(
      _agg_body,
      out_type=[jax.ShapeDtypeStruct((N, CK), _f32)] * NCK,
      mesh=_sc_mesh(),
      scratch_types=[
          pltpu.VMEM((NB, B), jnp.int32),
          pltpu.VMEM((NB, B), jnp.int32),
          pltpu.VMEM((D, B, CK), _f32),
          pltpu.SemaphoreType.DMA((D,)),
          pltpu.SemaphoreType.DMA((D,)),
          pltpu.VMEM_SHARED((ACC_ROWS, CK), _f32),
          pltpu.VMEM_SHARED((ACC_ROWS, CK), _f32),
      ],
      compiler_params=pltpu.CompilerParams(use_tc_tiling_on_sc=False),
  )
  return fn(*y_chunks, srcp, dstp)


def _deg_body(dstp, ones_h, zeros_h, degf, dst_v, buf, acc):
  c = lax.axis_index("c")
  s = lax.axis_index("s")

  @pl.when(c == 0)
  def _():
    pltpu.sync_copy(dstp.at[s], dst_v)
    pltpu.sync_copy(ones_h, buf)
    z0 = pl.multiple_of(s * RPT, 8)
    pltpu.sync_copy(zeros_h.at[pl.ds(z0, RPT)], acc.at[pl.ds(z0, RPT)])
    plsc.subcore_barrier()

    def body(j, carry):
      pltpu.sync_copy(buf, acc.at[dst_v.at[j]], add=True)
      return carry

    lax.fori_loop(0, NB, body, 0)
    plsc.subcore_barrier()
    _copy_rows(s, acc, degf)


def _deg_call(dstp, ones_h, zeros_h):
  fn = pl.kernel(
      _deg_body,
      out_type=jax.ShapeDtypeStruct((N, CK), _f32),
      mesh=_sc_mesh(),
      scratch_types=[
          pltpu.VMEM((NB, B), jnp.int32),
          pltpu.VMEM((B, CK), _f32),
          pltpu.VMEM_SHARED((ACC_ROWS, CK), _f32),
      ],
      compiler_params=pltpu.CompilerParams(use_tc_tiling_on_sc=False),
  )
  return fn(dstp, ones_h, zeros_h)


R0 = 1000


def _tc0_body(x_ref, w_ref, deg_ref, *y_refs):
  dinv = lax.rsqrt(deg_ref[:, 0:1] + 1.0)
  y = jnp.dot(x_ref[...] * dinv, w_ref[...], preferred_element_type=_f32)
  for k, yr in enumerate(y_refs):
    yr[...] = y[:, k * CK:(k + 1) * CK]


def _tc0(x, W, degf):
  return pl.pallas_call(
      _tc0_body,
      grid=(N // R0,),
      in_specs=[
          pl.BlockSpec((R0, CH0), lambda i: (i, 0)),
          pl.BlockSpec((CH0, H), lambda i: (0, 0)),
          pl.BlockSpec((R0, CK), lambda i: (i, 0)),
      ],
      out_specs=[pl.BlockSpec((R0, CK), lambda i: (i, 0))] * NCK,
      out_shape=[jax.ShapeDtypeStruct((N, CK), _f32)] * NCK,
  )(x, W, degf)


def _tcmid_body(*refs):
  a_refs = refs[:NCK]
  w_ref, deg_ref, b_ref = refs[NCK:NCK + 3]
  y_refs = refs[NCK + 3:]
  dinv = lax.rsqrt(deg_ref[:, 0:1] + 1.0)
  h = jnp.concatenate([a[...] for a in a_refs], axis=1)
  h = jnp.maximum(h * dinv + b_ref[...], 0.0)
  y = jnp.dot(h * dinv, w_ref[...], preferred_element_type=_f32)
  for k, yr in enumerate(y_refs):
    yr[...] = y[:, k * CK:(k + 1) * CK]


def _tcmid(acc, W, b, degf):
  return pl.pallas_call(
      _tcmid_body,
      grid=(N // R0,),
      in_specs=[pl.BlockSpec((R0, CK), lambda i: (i, 0))] * NCK + [
          pl.BlockSpec((H, H), lambda i: (0, 0)),
          pl.BlockSpec((R0, CK), lambda i: (i, 0)),
          pl.BlockSpec((1, H), lambda i: (0, 0)),
      ],
      out_specs=[pl.BlockSpec((R0, CK), lambda i: (i, 0))] * NCK,
      out_shape=[jax.ShapeDtypeStruct((N, CK), _f32)] * NCK,
  )(*acc, W, degf, b.reshape(1, H))


RF = 400
NGF = N // RF


def _fin_body(*refs):
  a_refs = refs[:NCK]
  (deg_ref, b_ref, batch_ref, desc_ref, wd_ref, bd_ref, wl_ref, bl_ref,
   out_ref, sums, counts) = refs[NCK:]
  i = pl.program_id(0)

  @pl.when(i == 0)
  def _():
    sums[...] = jnp.zeros_like(sums)
    counts[...] = jnp.zeros_like(counts)

  dinv = lax.rsqrt(deg_ref[:, 0:1] + 1.0)
  h = jnp.concatenate([a[...] for a in a_refs], axis=1)
  h = jnp.maximum(h * dinv + b_ref[...], 0.0)
  gids = lax.broadcasted_iota(jnp.int32, (RF, G), 1)
  P = (batch_ref[...] == gids).astype(_f32)
  sums[...] += lax.dot_general(P, h, (((0,), (0,)), ((), ())),
                               preferred_element_type=_f32)
  counts[...] += lax.dot_general(P, jnp.ones((RF, 1), _f32),
                                 (((0,), (0,)), ((), ())),
                                 preferred_element_type=_f32)

  @pl.when(i == NGF - 1)
  def _():
    gm = sums[...] / jnp.maximum(counts[...], 1.0)
    de = jnp.maximum(
        jnp.dot(desc_ref[...], wd_ref[...], preferred_element_type=_f32)
        + bd_ref[...], 0.0)
    z = jnp.concatenate([gm, de], axis=1)
    logit = jnp.dot(z, wl_ref[...], preferred_element_type=_f32) + bl_ref[...]
    out_ref[...] = jax.nn.sigmoid(logit)


def _tcfinal(acc, b, degf, batch, descriptors, Wd, bd, Wlin, blin):
  return pl.pallas_call(
      _fin_body,
      grid=(NGF,),
      in_specs=[pl.BlockSpec((RF, CK), lambda i: (i, 0))] * NCK + [
          pl.BlockSpec((RF, CK), lambda i: (i, 0)),
          pl.BlockSpec((1, H), lambda i: (0, 0)),
          pl.BlockSpec((RF, 1), lambda i: (i, 0)),
          pl.BlockSpec((G, DESC), lambda i: (0, 0)),
          pl.BlockSpec((DESC, H), lambda i: (0, 0)),
          pl.BlockSpec((1, H), lambda i: (0, 0)),
          pl.BlockSpec((2 * H, 1), lambda i: (0, 0)),
          pl.BlockSpec((1, 1), lambda i: (0, 0)),
      ],
      out_specs=pl.BlockSpec((G, 1), lambda i: (0, 0)),
      out_shape=jax.ShapeDtypeStruct((G, 1), _f32),
      scratch_shapes=[
          pltpu.VMEM((G, H), _f32),
          pltpu.VMEM((G, 1), _f32),
      ],
  )(*acc, degf, b.reshape(1, H), batch.reshape(N, 1), descriptors,
    Wd, bd.reshape(1, H), Wlin, blin.reshape(1, 1))


def kernel(x, edge_index, batch, descriptors,
           W0, b0, W1, b1, W2, b2, W3, b3, Wd, bd, Wlin, blin):
  src = edge_index[0].reshape(NT, EPT)
  dst = edge_index[1].reshape(NT, EPT)
  pad = EPP - EPT
  srcp = jnp.pad(src, ((0, 0), (0, pad)), constant_values=0).reshape(NT, NB, B)
  dstp = jnp.pad(dst, ((0, 0), (0, pad)), constant_values=N).reshape(NT, NB, B)
  ones_h = jnp.ones((B, CK), _f32)
  zeros_h = jnp.zeros((ACC_ROWS, CK), _f32)

  degf = _deg_call(dstp, ones_h, zeros_h)
  y = _tc0(x, W0, degf)
  bs = [b0, b1, b2, b3]
  Ws = [W1, W2, W3]
  for l in range(3):
    acc = _agg_call(y, srcp, dstp)
    y = _tcmid(acc, Ws[l], bs[l], degf)
  acc = _agg_call(y, srcp, dstp)
  out = _tcfinal(acc, bs[3], degf, batch, descriptors, Wd, bd, Wlin, blin)
  return out.reshape(-1)


# speedup vs baseline: 1.1811x; 1.0003x over previous
import jax
import jax.numpy as jnp
from jax import lax
from jax.experimental import pallas as pl
from jax.experimental.pallas import tpu as pltpu
from jax.experimental.pallas import tpu_sc as plsc

N = 10000
E = 160000
CH0 = 256
H = 512
G = 64
DESC = 128

CK = 64
NCK = H // CK
NT = 16
NC = 2
NPC = NCK // NC
EPT = E // NT
B = 128
D = 3
NGRP = (EPT + B * D - 1) // (B * D)
NB = NGRP * D
EPP = NB * B
RPT = 632
RPT_LAST = N - (NT - 1) * RPT
ACC_ROWS = NT * RPT

_f32 = jnp.float32


def _copy_rows(s, src_ref, dst_ref):
  r0 = pl.multiple_of(s * RPT, 8)

  @pl.when(s < NT - 1)
  def _():
    pltpu.sync_copy(src_ref.at[pl.ds(r0, RPT)], dst_ref.at[pl.ds(r0, RPT)])

  @pl.when(s == NT - 1)
  def _():
    base = (NT - 1) * RPT
    pltpu.sync_copy(src_ref.at[pl.ds(base, RPT_LAST)],
                    dst_ref.at[pl.ds(base, RPT_LAST)])


def _sc_mesh():
  return plsc.VectorSubcoreMesh(
      core_axis_name="c", subcore_axis_name="s",
      num_cores=NC, num_subcores=NT)


def _agg_body(*refs):
  ys = refs[:NCK]
  srcp, dstp = refs[NCK:NCK + 2]
  os_ = refs[NCK + 2:2 * NCK + 2]
  src_v, dst_v, bufs, gsems, ssems, acc, ysh = refs[2 * NCK + 2:]
  c = lax.axis_index("c")
  s = lax.axis_index("s")
  pltpu.sync_copy(srcp.at[s], src_v)
  pltpu.sync_copy(dstp.at[s], dst_v)

  def process(y_ref, o_ref):
    _copy_rows(s, y_ref, ysh)
    _copy_rows(s, y_ref, acc)
    plsc.subcore_barrier()

    def fire_gathers(base):
      return [pltpu.async_copy(
          ysh.at[src_v.at[base + k]], bufs.at[k], gsems.at[k])
          for k in range(D)]

    def fire_scatters(base, descs):
      for k in range(D):
        descs[k].wait()
        pltpu.async_copy(bufs.at[k], acc.at[dst_v.at[base + k]],
                         ssems.at[k], add=True)

    fire_scatters(0, fire_gathers(0))

    def body(g, carry):
      base = g * D
      descs = []
      for k in range(D):
        pltpu.make_async_copy(
            y_ref.at[pl.ds(0, B)], bufs.at[k], ssems.at[k]).wait()
        descs.append(pltpu.async_copy(
            ysh.at[src_v.at[base + k]], bufs.at[k], gsems.at[k]))
      fire_scatters(base, descs)
      return carry

    lax.fori_loop(1, NGRP, body, 0)
    for k in range(D):
      pltpu.make_async_copy(
          y_ref.at[pl.ds(0, B)], bufs.at[k], ssems.at[k]).wait()
    plsc.subcore_barrier()
    _copy_rows(s, acc, o_ref)
    plsc.subcore_barrier()

  @pl.when(c == 0)
  def _():
    for k in range(NPC):
      process(ys[k], os_[k])

  @pl.when(c == 1)
  def _():
    for k in range(NPC):
      process(ys[NPC + k], os_[NPC + k])


def _agg_call(y_chunks, srcp, dstp):
  fn = pl.kernel(
      _agg_body,
      out_type=[jax.ShapeDtypeStruct((N, CK), _f32)] * NCK,
      mesh=_sc_mesh(),
      scratch_types=[
          pltpu.VMEM((NB, B), jnp.int32),
          pltpu.VMEM((NB, B), jnp.int32),
          pltpu.VMEM((D, B, CK), _f32),
          pltpu.SemaphoreType.DMA((D,)),
          pltpu.SemaphoreType.DMA((D,)),
          pltpu.VMEM_SHARED((ACC_ROWS, CK), _f32),
          pltpu.VMEM_SHARED((ACC_ROWS, CK), _f32),
      ],
      compiler_params=pltpu.CompilerParams(use_tc_tiling_on_sc=False),
  )
  return fn(*y_chunks, srcp, dstp)


def _deg_body(dstp, ones_h, zeros_h, degf, dst_v, buf, acc):
  c = lax.axis_index("c")
  s = lax.axis_index("s")

  @pl.when(c == 0)
  def _():
    pltpu.sync_copy(dstp.at[s], dst_v)
    pltpu.sync_copy(ones_h, buf)
    z0 = pl.multiple_of(s * RPT, 8)
    pltpu.sync_copy(zeros_h.at[pl.ds(z0, RPT)], acc.at[pl.ds(z0, RPT)])
    plsc.subcore_barrier()

    def body(j, carry):
      pltpu.sync_copy(buf, acc.at[dst_v.at[j]], add=True)
      return carry

    lax.fori_loop(0, NB, body, 0)
    plsc.subcore_barrier()
    _copy_rows(s, acc, degf)


def _deg_call(dstp, ones_h, zeros_h):
  fn = pl.kernel(
      _deg_body,
      out_type=jax.ShapeDtypeStruct((N, CK), _f32),
      mesh=_sc_mesh(),
      scratch_types=[
          pltpu.VMEM((NB, B), jnp.int32),
          pltpu.VMEM((B, CK), _f32),
          pltpu.VMEM_SHARED((ACC_ROWS, CK), _f32),
      ],
      compiler_params=pltpu.CompilerParams(use_tc_tiling_on_sc=False),
  )
  return fn(dstp, ones_h, zeros_h)


R0 = 1000


def _tc0_body(x_ref, w_ref, deg_ref, *y_refs):
  dinv = lax.rsqrt(deg_ref[:, 0:1] + 1.0)
  y = jnp.dot(x_ref[...] * dinv, w_ref[...], preferred_element_type=_f32)
  for k, yr in enumerate(y_refs):
    yr[...] = y[:, k * CK:(k + 1) * CK]


def _tc0(x, W, degf):
  return pl.pallas_call(
      _tc0_body,
      grid=(N // R0,),
      in_specs=[
          pl.BlockSpec((R0, CH0), lambda i: (i, 0)),
          pl.BlockSpec((CH0, H), lambda i: (0, 0)),
          pl.BlockSpec((R0, CK), lambda i: (i, 0)),
      ],
      out_specs=[pl.BlockSpec((R0, CK), lambda i: (i, 0))] * NCK,
      out_shape=[jax.ShapeDtypeStruct((N, CK), _f32)] * NCK,
  )(x, W, degf)


def _tcmid_body(*refs):
  a_refs = refs[:NCK]
  w_ref, deg_ref, b_ref = refs[NCK:NCK + 3]
  y_refs = refs[NCK + 3:]
  dinv = lax.rsqrt(deg_ref[:, 0:1] + 1.0)
  h = jnp.concatenate([a[...] for a in a_refs], axis=1)
  h = jnp.maximum(h * dinv + b_ref[...], 0.0)
  y = jnp.dot(h * dinv, w_ref[...], preferred_element_type=_f32)
  for k, yr in enumerate(y_refs):
    yr[...] = y[:, k * CK:(k + 1) * CK]


def _tcmid(acc, W, b, degf):
  return pl.pallas_call(
      _tcmid_body,
      grid=(N // R0,),
      in_specs=[pl.BlockSpec((R0, CK), lambda i: (i, 0))] * NCK + [
          pl.BlockSpec((H, H), lambda i: (0, 0)),
          pl.BlockSpec((R0, CK), lambda i: (i, 0)),
          pl.BlockSpec((1, H), lambda i: (0, 0)),
      ],
      out_specs=[pl.BlockSpec((R0, CK), lambda i: (i, 0))] * NCK,
      out_shape=[jax.ShapeDtypeStruct((N, CK), _f32)] * NCK,
  )(*acc, W, degf, b.reshape(1, H))


RF = 400
NGF = N // RF


def _fin_body(*refs):
  a_refs = refs[:NCK]
  (deg_ref, b_ref, batch_ref, desc_ref, wd_ref, bd_ref, wl_ref, bl_ref,
   out_ref, sums, counts) = refs[NCK:]
  i = pl.program_id(0)

  @pl.when(i == 0)
  def _():
    sums[...] = jnp.zeros_like(sums)
    counts[...] = jnp.zeros_like(counts)

  dinv = lax.rsqrt(deg_ref[:, 0:1] + 1.0)
  h = jnp.concatenate([a[...] for a in a_refs], axis=1)
  h = jnp.maximum(h * dinv + b_ref[...], 0.0)
  gids = lax.broadcasted_iota(jnp.int32, (RF, G), 1)
  P = (batch_ref[...] == gids).astype(_f32)
  sums[...] += lax.dot_general(P, h, (((0,), (0,)), ((), ())),
                               preferred_element_type=_f32)
  counts[...] += lax.dot_general(P, jnp.ones((RF, 1), _f32),
                                 (((0,), (0,)), ((), ())),
                                 preferred_element_type=_f32)

  @pl.when(i == NGF - 1)
  def _():
    gm = sums[...] / jnp.maximum(counts[...], 1.0)
    de = jnp.maximum(
        jnp.dot(desc_ref[...], wd_ref[...], preferred_element_type=_f32)
        + bd_ref[...], 0.0)
    z = jnp.concatenate([gm, de], axis=1)
    logit = jnp.dot(z, wl_ref[...], preferred_element_type=_f32) + bl_ref[...]
    out_ref[...] = jax.nn.sigmoid(logit)


def _tcfinal(acc, b, degf, batch, descriptors, Wd, bd, Wlin, blin):
  return pl.pallas_call(
      _fin_body,
      grid=(NGF,),
      in_specs=[pl.BlockSpec((RF, CK), lambda i: (i, 0))] * NCK + [
          pl.BlockSpec((RF, CK), lambda i: (i, 0)),
          pl.BlockSpec((1, H), lambda i: (0, 0)),
          pl.BlockSpec((RF, 1), lambda i: (i, 0)),
          pl.BlockSpec((G, DESC), lambda i: (0, 0)),
          pl.BlockSpec((DESC, H), lambda i: (0, 0)),
          pl.BlockSpec((1, H), lambda i: (0, 0)),
          pl.BlockSpec((2 * H, 1), lambda i: (0, 0)),
          pl.BlockSpec((1, 1), lambda i: (0, 0)),
      ],
      out_specs=pl.BlockSpec((G, 1), lambda i: (0, 0)),
      out_shape=jax.ShapeDtypeStruct((G, 1), _f32),
      scratch_shapes=[
          pltpu.VMEM((G, H), _f32),
          pltpu.VMEM((G, 1), _f32),
      ],
  )(*acc, degf, b.reshape(1, H), batch.reshape(N, 1), descriptors,
    Wd, bd.reshape(1, H), Wlin, blin.reshape(1, 1))


def kernel(x, edge_index, batch, descriptors,
           W0, b0, W1, b1, W2, b2, W3, b3, Wd, bd, Wlin, blin):
  src = edge_index[0].reshape(NT, EPT)
  dst = edge_index[1].reshape(NT, EPT)
  pad = EPP - EPT
  srcp = jnp.pad(src, ((0, 0), (0, pad)), constant_values=0).reshape(NT, NB, B)
  dstp = jnp.pad(dst, ((0, 0), (0, pad)), constant_values=N).reshape(NT, NB, B)
  ones_h = jnp.ones((B, CK), _f32)
  zeros_h = jnp.zeros((ACC_ROWS, CK), _f32)

  degf = _deg_call(dstp, ones_h, zeros_h)
  y = _tc0(x, W0, degf)
  bs = [b0, b1, b2, b3]
  Ws = [W1, W2, W3]
  for l in range(3):
    acc = _agg_call(y, srcp, dstp)
    y = _tcmid(acc, Ws[l], bs[l], degf)
  acc = _agg_call(y, srcp, dstp)
  out = _tcfinal(acc, bs[3], degf, batch, descriptors, Wd, bd, Wlin, blin)
  return out.reshape(-1)


# speedup vs baseline: 1.8957x; 1.6050x over previous
import jax
import jax.numpy as jnp
from jax import lax
from jax.experimental import pallas as pl
from jax.experimental.pallas import tpu as pltpu
from jax.experimental.pallas import tpu_sc as plsc

N = 10000
E = 160000
CH0 = 256
H = 512
G = 64
DESC = 128

CK = 64
NCK = H // CK
NT = 16
NC = 2
NPC = NCK // NC
EPT = E // NT
B = 128
D = 3
NGRP = (EPT + B * D - 1) // (B * D)
NB = NGRP * D
EPP = NB * B
RPT = 632
RPT_LAST = N - (NT - 1) * RPT
ACC_ROWS = NT * RPT

_f32 = jnp.float32
_MSG = jnp.bfloat16


def _copy_rows(s, src_ref, dst_ref):
  r0 = pl.multiple_of(s * RPT, 8)

  @pl.when(s < NT - 1)
  def _():
    pltpu.sync_copy(src_ref.at[pl.ds(r0, RPT)], dst_ref.at[pl.ds(r0, RPT)])

  @pl.when(s == NT - 1)
  def _():
    base = (NT - 1) * RPT
    pltpu.sync_copy(src_ref.at[pl.ds(base, RPT_LAST)],
                    dst_ref.at[pl.ds(base, RPT_LAST)])


def _sc_mesh():
  return plsc.VectorSubcoreMesh(
      core_axis_name="c", subcore_axis_name="s",
      num_cores=NC, num_subcores=NT)


def _agg_body(*refs):
  ys = refs[:NCK]
  srcp, dstp = refs[NCK:NCK + 2]
  os_ = refs[NCK + 2:2 * NCK + 2]
  src_v, dst_v, bufs, gsems, ssems, acc, ysh = refs[2 * NCK + 2:]
  c = lax.axis_index("c")
  s = lax.axis_index("s")
  pltpu.sync_copy(srcp.at[s], src_v)
  pltpu.sync_copy(dstp.at[s], dst_v)

  def process(y_ref, o_ref):
    _copy_rows(s, y_ref, ysh)
    _copy_rows(s, y_ref, acc)
    plsc.subcore_barrier()

    def fire_gathers(base):
      return [pltpu.async_copy(
          ysh.at[src_v.at[base + k]], bufs.at[k], gsems.at[k])
          for k in range(D)]

    def fire_scatters(base, descs):
      for k in range(D):
        descs[k].wait()
        pltpu.async_copy(bufs.at[k], acc.at[dst_v.at[base + k]],
                         ssems.at[k], add=True)

    fire_scatters(0, fire_gathers(0))

    def body(g, carry):
      base = g * D
      descs = []
      for k in range(D):
        pltpu.make_async_copy(
            y_ref.at[pl.ds(0, B)], bufs.at[k], ssems.at[k]).wait()
        descs.append(pltpu.async_copy(
            ysh.at[src_v.at[base + k]], bufs.at[k], gsems.at[k]))
      fire_scatters(base, descs)
      return carry

    lax.fori_loop(1, NGRP, body, 0)
    for k in range(D):
      pltpu.make_async_copy(
          y_ref.at[pl.ds(0, B)], bufs.at[k], ssems.at[k]).wait()
    plsc.subcore_barrier()
    _copy_rows(s, acc, o_ref)
    plsc.subcore_barrier()

  @pl.when(c == 0)
  def _():
    for k in range(NPC):
      process(ys[k], os_[k])

  @pl.when(c == 1)
  def _():
    for k in range(NPC):
      process(ys[NPC + k], os_[NPC + k])


def _agg_call(y_chunks, srcp, dstp):
  fn = pl.kernel(
      _agg_body,
      out_type=[jax.ShapeDtypeStruct((N, CK), _MSG)] * NCK,
      mesh=_sc_mesh(),
      scratch_types=[
          pltpu.VMEM((NB, B), jnp.int32),
          pltpu.VMEM((NB, B), jnp.int32),
          pltpu.VMEM((D, B, CK), _MSG),
          pltpu.SemaphoreType.DMA((D,)),
          pltpu.SemaphoreType.DMA((D,)),
          pltpu.VMEM_SHARED((ACC_ROWS, CK), _MSG),
          pltpu.VMEM_SHARED((ACC_ROWS, CK), _MSG),
      ],
      compiler_params=pltpu.CompilerParams(use_tc_tiling_on_sc=False),
  )
  return fn(*y_chunks, srcp, dstp)


def _deg_body(dstp, ones_h, zeros_h, degf, dst_v, buf, acc):
  c = lax.axis_index("c")
  s = lax.axis_index("s")

  @pl.when(c == 0)
  def _():
    pltpu.sync_copy(dstp.at[s], dst_v)
    pltpu.sync_copy(ones_h, buf)
    z0 = pl.multiple_of(s * RPT, 8)
    pltpu.sync_copy(zeros_h.at[pl.ds(z0, RPT)], acc.at[pl.ds(z0, RPT)])
    plsc.subcore_barrier()

    def body(j, carry):
      pltpu.sync_copy(buf, acc.at[dst_v.at[j]], add=True)
      return carry

    lax.fori_loop(0, NB, body, 0)
    plsc.subcore_barrier()
    _copy_rows(s, acc, degf)


def _deg_call(dstp, ones_h, zeros_h):
  fn = pl.kernel(
      _deg_body,
      out_type=jax.ShapeDtypeStruct((N, CK), _f32),
      mesh=_sc_mesh(),
      scratch_types=[
          pltpu.VMEM((NB, B), jnp.int32),
          pltpu.VMEM((B, CK), _f32),
          pltpu.VMEM_SHARED((ACC_ROWS, CK), _f32),
      ],
      compiler_params=pltpu.CompilerParams(use_tc_tiling_on_sc=False),
  )
  return fn(dstp, ones_h, zeros_h)


R0 = 1000


def _tc0_body(x_ref, w_ref, deg_ref, *y_refs):
  dinv = lax.rsqrt(deg_ref[:, 0:1] + 1.0)
  y = jnp.dot(x_ref[...] * dinv, w_ref[...], preferred_element_type=_f32)
  for k, yr in enumerate(y_refs):
    yr[...] = y[:, k * CK:(k + 1) * CK].astype(_MSG)


def _tc0(x, W, degf):
  return pl.pallas_call(
      _tc0_body,
      grid=(N // R0,),
      in_specs=[
          pl.BlockSpec((R0, CH0), lambda i: (i, 0)),
          pl.BlockSpec((CH0, H), lambda i: (0, 0)),
          pl.BlockSpec((R0, CK), lambda i: (i, 0)),
      ],
      out_specs=[pl.BlockSpec((R0, CK), lambda i: (i, 0))] * NCK,
      out_shape=[jax.ShapeDtypeStruct((N, CK), _MSG)] * NCK,
  )(x, W, degf)


def _tcmid_body(*refs):
  a_refs = refs[:NCK]
  w_ref, deg_ref, b_ref = refs[NCK:NCK + 3]
  y_refs = refs[NCK + 3:]
  dinv = lax.rsqrt(deg_ref[:, 0:1] + 1.0)
  h = jnp.concatenate([a[...].astype(_f32) for a in a_refs], axis=1)
  h = jnp.maximum(h * dinv + b_ref[...], 0.0)
  y = jnp.dot(h * dinv, w_ref[...], preferred_element_type=_f32)
  for k, yr in enumerate(y_refs):
    yr[...] = y[:, k * CK:(k + 1) * CK].astype(_MSG)


def _tcmid(acc, W, b, degf):
  return pl.pallas_call(
      _tcmid_body,
      grid=(N // R0,),
      in_specs=[pl.BlockSpec((R0, CK), lambda i: (i, 0))] * NCK + [
          pl.BlockSpec((H, H), lambda i: (0, 0)),
          pl.BlockSpec((R0, CK), lambda i: (i, 0)),
          pl.BlockSpec((1, H), lambda i: (0, 0)),
      ],
      out_specs=[pl.BlockSpec((R0, CK), lambda i: (i, 0))] * NCK,
      out_shape=[jax.ShapeDtypeStruct((N, CK), _MSG)] * NCK,
  )(*acc, W, degf, b.reshape(1, H))


RF = 400
NGF = N // RF


def _fin_body(*refs):
  a_refs = refs[:NCK]
  (deg_ref, b_ref, batch_ref, desc_ref, wd_ref, bd_ref, wl_ref, bl_ref,
   out_ref, sums, counts) = refs[NCK:]
  i = pl.program_id(0)

  @pl.when(i == 0)
  def _():
    sums[...] = jnp.zeros_like(sums)
    counts[...] = jnp.zeros_like(counts)

  dinv = lax.rsqrt(deg_ref[:, 0:1] + 1.0)
  h = jnp.concatenate([a[...].astype(_f32) for a in a_refs], axis=1)
  h = jnp.maximum(h * dinv + b_ref[...], 0.0)
  gids = lax.broadcasted_iota(jnp.int32, (RF, G), 1)
  P = (batch_ref[...] == gids).astype(_f32)
  sums[...] += lax.dot_general(P, h, (((0,), (0,)), ((), ())),
                               preferred_element_type=_f32)
  counts[...] += lax.dot_general(P, jnp.ones((RF, 1), _f32),
                                 (((0,), (0,)), ((), ())),
                                 preferred_element_type=_f32)

  @pl.when(i == NGF - 1)
  def _():
    gm = sums[...] / jnp.maximum(counts[...], 1.0)
    de = jnp.maximum(
        jnp.dot(desc_ref[...], wd_ref[...], preferred_element_type=_f32)
        + bd_ref[...], 0.0)
    z = jnp.concatenate([gm, de], axis=1)
    logit = jnp.dot(z, wl_ref[...], preferred_element_type=_f32) + bl_ref[...]
    out_ref[...] = jax.nn.sigmoid(logit)


def _tcfinal(acc, b, degf, batch, descriptors, Wd, bd, Wlin, blin):
  return pl.pallas_call(
      _fin_body,
      grid=(NGF,),
      in_specs=[pl.BlockSpec((RF, CK), lambda i: (i, 0))] * NCK + [
          pl.BlockSpec((RF, CK), lambda i: (i, 0)),
          pl.BlockSpec((1, H), lambda i: (0, 0)),
          pl.BlockSpec((RF, 1), lambda i: (i, 0)),
          pl.BlockSpec((G, DESC), lambda i: (0, 0)),
          pl.BlockSpec((DESC, H), lambda i: (0, 0)),
          pl.BlockSpec((1, H), lambda i: (0, 0)),
          pl.BlockSpec((2 * H, 1), lambda i: (0, 0)),
          pl.BlockSpec((1, 1), lambda i: (0, 0)),
      ],
      out_specs=pl.BlockSpec((G, 1), lambda i: (0, 0)),
      out_shape=jax.ShapeDtypeStruct((G, 1), _f32),
      scratch_shapes=[
          pltpu.VMEM((G, H), _f32),
          pltpu.VMEM((G, 1), _f32),
      ],
  )(*acc, degf, b.reshape(1, H), batch.reshape(N, 1), descriptors,
    Wd, bd.reshape(1, H), Wlin, blin.reshape(1, 1))


def kernel(x, edge_index, batch, descriptors,
           W0, b0, W1, b1, W2, b2, W3, b3, Wd, bd, Wlin, blin):
  src = edge_index[0].reshape(NT, EPT)
  dst = edge_index[1].reshape(NT, EPT)
  pad = EPP - EPT
  srcp = jnp.pad(src, ((0, 0), (0, pad)), constant_values=0).reshape(NT, NB, B)
  dstp = jnp.pad(dst, ((0, 0), (0, pad)), constant_values=N).reshape(NT, NB, B)
  ones_h = jnp.ones((B, CK), _f32)
  zeros_h = jnp.zeros((ACC_ROWS, CK), _f32)

  degf = _deg_call(dstp, ones_h, zeros_h)
  y = _tc0(x, W0, degf)
  bs = [b0, b1, b2, b3]
  Ws = [W1, W2, W3]
  for l in range(3):
    acc = _agg_call(y, srcp, dstp)
    y = _tcmid(acc, Ws[l], bs[l], degf)
  acc = _agg_call(y, srcp, dstp)
  out = _tcfinal(acc, bs[3], degf, batch, descriptors, Wd, bd, Wlin, blin)
  return out.reshape(-1)


# speedup vs baseline: 1.9049x; 1.0049x over previous
import jax
import jax.numpy as jnp
from jax import lax
from jax.experimental import pallas as pl
from jax.experimental.pallas import tpu as pltpu
from jax.experimental.pallas import tpu_sc as plsc

N = 10000
E = 160000
CH0 = 256
H = 512
G = 64
DESC = 128

CK = 128
NCK = H // CK
NT = 16
NC = 2
NPC = NCK // NC
EPT = E // NT
B = 128
D = 3
NGRP = (EPT + B * D - 1) // (B * D)
NB = NGRP * D
EPP = NB * B
RPT = 632
RPT_LAST = N - (NT - 1) * RPT
ACC_ROWS = NT * RPT

_f32 = jnp.float32
_MSG = jnp.bfloat16


def _copy_rows(s, src_ref, dst_ref):
  r0 = pl.multiple_of(s * RPT, 8)

  @pl.when(s < NT - 1)
  def _():
    pltpu.sync_copy(src_ref.at[pl.ds(r0, RPT)], dst_ref.at[pl.ds(r0, RPT)])

  @pl.when(s == NT - 1)
  def _():
    base = (NT - 1) * RPT
    pltpu.sync_copy(src_ref.at[pl.ds(base, RPT_LAST)],
                    dst_ref.at[pl.ds(base, RPT_LAST)])


def _sc_mesh():
  return plsc.VectorSubcoreMesh(
      core_axis_name="c", subcore_axis_name="s",
      num_cores=NC, num_subcores=NT)


def _agg_body(*refs):
  ys = refs[:NCK]
  srcp, dstp = refs[NCK:NCK + 2]
  os_ = refs[NCK + 2:2 * NCK + 2]
  src_v, dst_v, bufs, gsems, ssems, acc, ysh = refs[2 * NCK + 2:]
  c = lax.axis_index("c")
  s = lax.axis_index("s")
  pltpu.sync_copy(srcp.at[s], src_v)
  pltpu.sync_copy(dstp.at[s], dst_v)

  def process(y_ref, o_ref):
    _copy_rows(s, y_ref, ysh)
    _copy_rows(s, y_ref, acc)
    plsc.subcore_barrier()

    def fire_gathers(base):
      return [pltpu.async_copy(
          ysh.at[src_v.at[base + k]], bufs.at[k], gsems.at[k])
          for k in range(D)]

    def fire_scatters(base, descs):
      for k in range(D):
        descs[k].wait()
        pltpu.async_copy(bufs.at[k], acc.at[dst_v.at[base + k]],
                         ssems.at[k], add=True)

    fire_scatters(0, fire_gathers(0))

    def body(g, carry):
      base = g * D
      descs = []
      for k in range(D):
        pltpu.make_async_copy(
            y_ref.at[pl.ds(0, B)], bufs.at[k], ssems.at[k]).wait()
        descs.append(pltpu.async_copy(
            ysh.at[src_v.at[base + k]], bufs.at[k], gsems.at[k]))
      fire_scatters(base, descs)
      return carry

    lax.fori_loop(1, NGRP, body, 0)
    for k in range(D):
      pltpu.make_async_copy(
          y_ref.at[pl.ds(0, B)], bufs.at[k], ssems.at[k]).wait()
    plsc.subcore_barrier()
    _copy_rows(s, acc, o_ref)
    plsc.subcore_barrier()

  @pl.when(c == 0)
  def _():
    for k in range(NPC):
      process(ys[k], os_[k])

  @pl.when(c == 1)
  def _():
    for k in range(NPC):
      process(ys[NPC + k], os_[NPC + k])


def _agg_call(y_chunks, srcp, dstp):
  fn = pl.kernel(
      _agg_body,
      out_type=[jax.ShapeDtypeStruct((N, CK), _MSG)] * NCK,
      mesh=_sc_mesh(),
      scratch_types=[
          pltpu.VMEM((NB, B), jnp.int32),
          pltpu.VMEM((NB, B), jnp.int32),
          pltpu.VMEM((D, B, CK), _MSG),
          pltpu.SemaphoreType.DMA((D,)),
          pltpu.SemaphoreType.DMA((D,)),
          pltpu.VMEM_SHARED((ACC_ROWS, CK), _MSG),
          pltpu.VMEM_SHARED((ACC_ROWS, CK), _MSG),
      ],
      compiler_params=pltpu.CompilerParams(use_tc_tiling_on_sc=False),
  )
  return fn(*y_chunks, srcp, dstp)


def _deg_body(dstp, ones_h, zeros_h, degf, dst_v, buf, acc):
  c = lax.axis_index("c")
  s = lax.axis_index("s")

  @pl.when(c == 0)
  def _():
    pltpu.sync_copy(dstp.at[s], dst_v)
    pltpu.sync_copy(ones_h, buf)
    z0 = pl.multiple_of(s * RPT, 8)
    pltpu.sync_copy(zeros_h.at[pl.ds(z0, RPT)], acc.at[pl.ds(z0, RPT)])
    plsc.subcore_barrier()

    def body(j, carry):
      pltpu.sync_copy(buf, acc.at[dst_v.at[j]], add=True)
      return carry

    lax.fori_loop(0, NB, body, 0)
    plsc.subcore_barrier()
    _copy_rows(s, acc, degf)


def _deg_call(dstp, ones_h, zeros_h):
  fn = pl.kernel(
      _deg_body,
      out_type=jax.ShapeDtypeStruct((N, CK), _f32),
      mesh=_sc_mesh(),
      scratch_types=[
          pltpu.VMEM((NB, B), jnp.int32),
          pltpu.VMEM((B, CK), _f32),
          pltpu.VMEM_SHARED((ACC_ROWS, CK), _f32),
      ],
      compiler_params=pltpu.CompilerParams(use_tc_tiling_on_sc=False),
  )
  return fn(dstp, ones_h, zeros_h)


R0 = 1000


def _tc0_body(x_ref, w_ref, deg_ref, *y_refs):
  dinv = lax.rsqrt(deg_ref[:, 0:1] + 1.0)
  y = jnp.dot(x_ref[...] * dinv, w_ref[...], preferred_element_type=_f32)
  for k, yr in enumerate(y_refs):
    yr[...] = y[:, k * CK:(k + 1) * CK].astype(_MSG)


def _tc0(x, W, degf):
  return pl.pallas_call(
      _tc0_body,
      grid=(N // R0,),
      in_specs=[
          pl.BlockSpec((R0, CH0), lambda i: (i, 0)),
          pl.BlockSpec((CH0, H), lambda i: (0, 0)),
          pl.BlockSpec((R0, CK), lambda i: (i, 0)),
      ],
      out_specs=[pl.BlockSpec((R0, CK), lambda i: (i, 0))] * NCK,
      out_shape=[jax.ShapeDtypeStruct((N, CK), _MSG)] * NCK,
  )(x, W, degf)


def _tcmid_body(*refs):
  a_refs = refs[:NCK]
  w_ref, deg_ref, b_ref = refs[NCK:NCK + 3]
  y_refs = refs[NCK + 3:]
  dinv = lax.rsqrt(deg_ref[:, 0:1] + 1.0)
  h = jnp.concatenate([a[...].astype(_f32) for a in a_refs], axis=1)
  h = jnp.maximum(h * dinv + b_ref[...], 0.0)
  y = jnp.dot(h * dinv, w_ref[...], preferred_element_type=_f32)
  for k, yr in enumerate(y_refs):
    yr[...] = y[:, k * CK:(k + 1) * CK].astype(_MSG)


def _tcmid(acc, W, b, degf):
  return pl.pallas_call(
      _tcmid_body,
      grid=(N // R0,),
      in_specs=[pl.BlockSpec((R0, CK), lambda i: (i, 0))] * NCK + [
          pl.BlockSpec((H, H), lambda i: (0, 0)),
          pl.BlockSpec((R0, CK), lambda i: (i, 0)),
          pl.BlockSpec((1, H), lambda i: (0, 0)),
      ],
      out_specs=[pl.BlockSpec((R0, CK), lambda i: (i, 0))] * NCK,
      out_shape=[jax.ShapeDtypeStruct((N, CK), _MSG)] * NCK,
  )(*acc, W, degf, b.reshape(1, H))


RF = 400
NGF = N // RF


def _fin_body(*refs):
  a_refs = refs[:NCK]
  (deg_ref, b_ref, batch_ref, desc_ref, wd_ref, bd_ref, wl_ref, bl_ref,
   out_ref, sums, counts) = refs[NCK:]
  i = pl.program_id(0)

  @pl.when(i == 0)
  def _():
    sums[...] = jnp.zeros_like(sums)
    counts[...] = jnp.zeros_like(counts)

  dinv = lax.rsqrt(deg_ref[:, 0:1] + 1.0)
  h = jnp.concatenate([a[...].astype(_f32) for a in a_refs], axis=1)
  h = jnp.maximum(h * dinv + b_ref[...], 0.0)
  gids = lax.broadcasted_iota(jnp.int32, (RF, G), 1)
  P = (batch_ref[...] == gids).astype(_f32)
  sums[...] += lax.dot_general(P, h, (((0,), (0,)), ((), ())),
                               preferred_element_type=_f32)
  counts[...] += lax.dot_general(P, jnp.ones((RF, 1), _f32),
                                 (((0,), (0,)), ((), ())),
                                 preferred_element_type=_f32)

  @pl.when(i == NGF - 1)
  def _():
    gm = sums[...] / jnp.maximum(counts[...], 1.0)
    de = jnp.maximum(
        jnp.dot(desc_ref[...], wd_ref[...], preferred_element_type=_f32)
        + bd_ref[...], 0.0)
    z = jnp.concatenate([gm, de], axis=1)
    logit = jnp.dot(z, wl_ref[...], preferred_element_type=_f32) + bl_ref[...]
    out_ref[...] = jax.nn.sigmoid(logit)


def _tcfinal(acc, b, degf, batch, descriptors, Wd, bd, Wlin, blin):
  return pl.pallas_call(
      _fin_body,
      grid=(NGF,),
      in_specs=[pl.BlockSpec((RF, CK), lambda i: (i, 0))] * NCK + [
          pl.BlockSpec((RF, CK), lambda i: (i, 0)),
          pl.BlockSpec((1, H), lambda i: (0, 0)),
          pl.BlockSpec((RF, 1), lambda i: (i, 0)),
          pl.BlockSpec((G, DESC), lambda i: (0, 0)),
          pl.BlockSpec((DESC, H), lambda i: (0, 0)),
          pl.BlockSpec((1, H), lambda i: (0, 0)),
          pl.BlockSpec((2 * H, 1), lambda i: (0, 0)),
          pl.BlockSpec((1, 1), lambda i: (0, 0)),
      ],
      out_specs=pl.BlockSpec((G, 1), lambda i: (0, 0)),
      out_shape=jax.ShapeDtypeStruct((G, 1), _f32),
      scratch_shapes=[
          pltpu.VMEM((G, H), _f32),
          pltpu.VMEM((G, 1), _f32),
      ],
  )(*acc, degf, b.reshape(1, H), batch.reshape(N, 1), descriptors,
    Wd, bd.reshape(1, H), Wlin, blin.reshape(1, 1))


def kernel(x, edge_index, batch, descriptors,
           W0, b0, W1, b1, W2, b2, W3, b3, Wd, bd, Wlin, blin):
  src = edge_index[0].reshape(NT, EPT)
  dst = edge_index[1].reshape(NT, EPT)
  pad = EPP - EPT
  srcp = jnp.pad(src, ((0, 0), (0, pad)), constant_values=0).reshape(NT, NB, B)
  dstp = jnp.pad(dst, ((0, 0), (0, pad)), constant_values=N).reshape(NT, NB, B)
  ones_h = jnp.ones((B, CK), _f32)
  zeros_h = jnp.zeros((ACC_ROWS, CK), _f32)

  degf = _deg_call(dstp, ones_h, zeros_h)
  y = _tc0(x, W0, degf)
  bs = [b0, b1, b2, b3]
  Ws = [W1, W2, W3]
  for l in range(3):
    acc = _agg_call(y, srcp, dstp)
    y = _tcmid(acc, Ws[l], bs[l], degf)
  acc = _agg_call(y, srcp, dstp)
  out = _tcfinal(acc, bs[3], degf, batch, descriptors, Wd, bd, Wlin, blin)
  return out.reshape(-1)


# speedup vs baseline: 1.9897x; 1.0445x over previous
import jax
import jax.numpy as jnp
from jax import lax
from jax.experimental import pallas as pl
from jax.experimental.pallas import tpu as pltpu
from jax.experimental.pallas import tpu_sc as plsc

N = 10000
E = 160000
CH0 = 256
H = 512
G = 64
DESC = 128

CK = 128
NCK = H // CK
NT = 16
NC = 2
NPC = NCK // NC
EPT = E // NT
B = 128
D = 3
NGRP = (EPT + B * D - 1) // (B * D)
NB = NGRP * D
EPP = NB * B
RPT = 632
RPT_LAST = N - (NT - 1) * RPT
ACC_ROWS = NT * RPT

_f32 = jnp.float32
_MSG = jnp.bfloat16


def _copy_rows(s, src_ref, dst_ref):
  r0 = pl.multiple_of(s * RPT, 8)

  @pl.when(s < NT - 1)
  def _():
    pltpu.sync_copy(src_ref.at[pl.ds(r0, RPT)], dst_ref.at[pl.ds(r0, RPT)])

  @pl.when(s == NT - 1)
  def _():
    base = (NT - 1) * RPT
    pltpu.sync_copy(src_ref.at[pl.ds(base, RPT_LAST)],
                    dst_ref.at[pl.ds(base, RPT_LAST)])


def _sc_mesh():
  return plsc.VectorSubcoreMesh(
      core_axis_name="c", subcore_axis_name="s",
      num_cores=NC, num_subcores=NT)


def _agg_body(*refs):
  ys = refs[:NCK]
  srcp, dstp = refs[NCK:NCK + 2]
  os_ = refs[NCK + 2:2 * NCK + 2]
  src_v, dst_v, bufs, gsems, ssems, acc, ysh = refs[2 * NCK + 2:]
  c = lax.axis_index("c")
  s = lax.axis_index("s")
  pltpu.sync_copy(srcp.at[s], src_v)
  pltpu.sync_copy(dstp.at[s], dst_v)

  def process(y_ref, o_ref):
    _copy_rows(s, y_ref, ysh)
    _copy_rows(s, y_ref, acc)
    plsc.subcore_barrier()

    def fire_gathers(base):
      return [pltpu.async_copy(
          ysh.at[src_v.at[base + k]], bufs.at[k], gsems.at[k])
          for k in range(D)]

    def fire_scatters(base, descs):
      for k in range(D):
        descs[k].wait()
        pltpu.async_copy(bufs.at[k], acc.at[dst_v.at[base + k]],
                         ssems.at[k], add=True)

    fire_scatters(0, fire_gathers(0))

    def body(g, carry):
      base = g * D
      descs = []
      for k in range(D):
        pltpu.make_async_copy(
            y_ref.at[pl.ds(0, B)], bufs.at[k], ssems.at[k]).wait()
        descs.append(pltpu.async_copy(
            ysh.at[src_v.at[base + k]], bufs.at[k], gsems.at[k]))
      fire_scatters(base, descs)
      return carry

    lax.fori_loop(1, NGRP, body, 0)
    for k in range(D):
      pltpu.make_async_copy(
          y_ref.at[pl.ds(0, B)], bufs.at[k], ssems.at[k]).wait()
    plsc.subcore_barrier()
    _copy_rows(s, acc, o_ref)
    plsc.subcore_barrier()

  @pl.when(c == 0)
  def _():
    for k in range(NPC):
      process(ys[k], os_[k])

  @pl.when(c == 1)
  def _():
    for k in range(NPC):
      process(ys[NPC + k], os_[NPC + k])


def _agg_call(y_chunks, srcp, dstp):
  fn = pl.kernel(
      _agg_body,
      out_type=[jax.ShapeDtypeStruct((N, CK), _MSG)] * NCK,
      mesh=_sc_mesh(),
      scratch_types=[
          pltpu.VMEM((NB, B), jnp.int32),
          pltpu.VMEM((NB, B), jnp.int32),
          pltpu.VMEM((D, B, CK), _MSG),
          pltpu.SemaphoreType.DMA((D,)),
          pltpu.SemaphoreType.DMA((D,)),
          pltpu.VMEM_SHARED((ACC_ROWS, CK), _MSG),
          pltpu.VMEM_SHARED((ACC_ROWS, CK), _MSG),
      ],
      compiler_params=pltpu.CompilerParams(use_tc_tiling_on_sc=False),
  )
  return fn(*y_chunks, srcp, dstp)


CKD = 32


def _deg_body(dstp, ones_h, zeros_h, degf, dst_v, buf, acc):
  c = lax.axis_index("c")
  s = lax.axis_index("s")

  @pl.when(c == 0)
  def _():
    pltpu.sync_copy(dstp.at[s], dst_v)
    pltpu.sync_copy(ones_h, buf)
    z0 = pl.multiple_of(s * RPT, 8)
    pltpu.sync_copy(zeros_h.at[pl.ds(z0, RPT)], acc.at[pl.ds(z0, RPT)])
    plsc.subcore_barrier()

    def body(j, carry):
      pltpu.sync_copy(buf, acc.at[dst_v.at[j]], add=True)
      return carry

    lax.fori_loop(0, NB, body, 0)
    plsc.subcore_barrier()
    _copy_rows(s, acc, degf)


def _deg_call(dstp, ones_h, zeros_h):
  fn = pl.kernel(
      _deg_body,
      out_type=jax.ShapeDtypeStruct((N, CKD), _MSG),
      mesh=_sc_mesh(),
      scratch_types=[
          pltpu.VMEM((NB, B), jnp.int32),
          pltpu.VMEM((B, CKD), _MSG),
          pltpu.VMEM_SHARED((ACC_ROWS, CKD), _MSG),
      ],
      compiler_params=pltpu.CompilerParams(use_tc_tiling_on_sc=False),
  )
  return fn(dstp, ones_h, zeros_h)


R0 = 1000


def _tc0_body(x_ref, w_ref, deg_ref, *y_refs):
  dinv = lax.rsqrt(deg_ref[:, 0:1].astype(_f32) + 1.0)
  y = jnp.dot((x_ref[...] * dinv).astype(_MSG), w_ref[...],
              preferred_element_type=_f32)
  for k, yr in enumerate(y_refs):
    yr[...] = y[:, k * CK:(k + 1) * CK].astype(_MSG)


def _tc0(x, W, degf):
  return pl.pallas_call(
      _tc0_body,
      grid=(N // R0,),
      in_specs=[
          pl.BlockSpec((R0, CH0), lambda i: (i, 0)),
          pl.BlockSpec((CH0, H), lambda i: (0, 0)),
          pl.BlockSpec((R0, CKD), lambda i: (i, 0)),
      ],
      out_specs=[pl.BlockSpec((R0, CK), lambda i: (i, 0))] * NCK,
      out_shape=[jax.ShapeDtypeStruct((N, CK), _MSG)] * NCK,
  )(x, W.astype(_MSG), degf)


def _tcmid_body(*refs):
  a_refs = refs[:NCK]
  w_ref, deg_ref, b_ref = refs[NCK:NCK + 3]
  y_refs = refs[NCK + 3:]
  dinv = lax.rsqrt(deg_ref[:, 0:1].astype(_f32) + 1.0)
  h = jnp.concatenate([a[...].astype(_f32) for a in a_refs], axis=1)
  h = jnp.maximum(h * dinv + b_ref[...], 0.0)
  y = jnp.dot((h * dinv).astype(_MSG), w_ref[...],
              preferred_element_type=_f32)
  for k, yr in enumerate(y_refs):
    yr[...] = y[:, k * CK:(k + 1) * CK].astype(_MSG)


def _tcmid(acc, W, b, degf):
  return pl.pallas_call(
      _tcmid_body,
      grid=(N // R0,),
      in_specs=[pl.BlockSpec((R0, CK), lambda i: (i, 0))] * NCK + [
          pl.BlockSpec((H, H), lambda i: (0, 0)),
          pl.BlockSpec((R0, CKD), lambda i: (i, 0)),
          pl.BlockSpec((1, H), lambda i: (0, 0)),
      ],
      out_specs=[pl.BlockSpec((R0, CK), lambda i: (i, 0))] * NCK,
      out_shape=[jax.ShapeDtypeStruct((N, CK), _MSG)] * NCK,
  )(*acc, W.astype(_MSG), degf, b.reshape(1, H))


RF = 400
NGF = N // RF


def _fin_body(*refs):
  a_refs = refs[:NCK]
  (deg_ref, b_ref, batch_ref, desc_ref, wd_ref, bd_ref, wl_ref, bl_ref,
   out_ref, sums, counts) = refs[NCK:]
  i = pl.program_id(0)

  @pl.when(i == 0)
  def _():
    sums[...] = jnp.zeros_like(sums)
    counts[...] = jnp.zeros_like(counts)

  dinv = lax.rsqrt(deg_ref[:, 0:1].astype(_f32) + 1.0)
  h = jnp.concatenate([a[...].astype(_f32) for a in a_refs], axis=1)
  h = jnp.maximum(h * dinv + b_ref[...], 0.0)
  gids = lax.broadcasted_iota(jnp.int32, (RF, G), 1)
  P = (batch_ref[...] == gids).astype(_f32)
  sums[...] += lax.dot_general(P, h, (((0,), (0,)), ((), ())),
                               preferred_element_type=_f32)
  counts[...] += lax.dot_general(P, jnp.ones((RF, 1), _f32),
                                 (((0,), (0,)), ((), ())),
                                 preferred_element_type=_f32)

  @pl.when(i == NGF - 1)
  def _():
    gm = sums[...] / jnp.maximum(counts[...], 1.0)
    de = jnp.maximum(
        jnp.dot(desc_ref[...], wd_ref[...], preferred_element_type=_f32)
        + bd_ref[...], 0.0)
    z = jnp.concatenate([gm, de], axis=1)
    logit = jnp.dot(z, wl_ref[...], preferred_element_type=_f32) + bl_ref[...]
    out_ref[...] = jax.nn.sigmoid(logit)


def _tcfinal(acc, b, degf, batch, descriptors, Wd, bd, Wlin, blin):
  return pl.pallas_call(
      _fin_body,
      grid=(NGF,),
      in_specs=[pl.BlockSpec((RF, CK), lambda i: (i, 0))] * NCK + [
          pl.BlockSpec((RF, CKD), lambda i: (i, 0)),
          pl.BlockSpec((1, H), lambda i: (0, 0)),
          pl.BlockSpec((RF, 1), lambda i: (i, 0)),
          pl.BlockSpec((G, DESC), lambda i: (0, 0)),
          pl.BlockSpec((DESC, H), lambda i: (0, 0)),
          pl.BlockSpec((1, H), lambda i: (0, 0)),
          pl.BlockSpec((2 * H, 1), lambda i: (0, 0)),
          pl.BlockSpec((1, 1), lambda i: (0, 0)),
      ],
      out_specs=pl.BlockSpec((G, 1), lambda i: (0, 0)),
      out_shape=jax.ShapeDtypeStruct((G, 1), _f32),
      scratch_shapes=[
          pltpu.VMEM((G, H), _f32),
          pltpu.VMEM((G, 1), _f32),
      ],
  )(*acc, degf, b.reshape(1, H), batch.reshape(N, 1), descriptors,
    Wd, bd.reshape(1, H), Wlin, blin.reshape(1, 1))


def kernel(x, edge_index, batch, descriptors,
           W0, b0, W1, b1, W2, b2, W3, b3, Wd, bd, Wlin, blin):
  src = edge_index[0].reshape(NT, EPT)
  dst = edge_index[1].reshape(NT, EPT)
  pad = EPP - EPT
  srcp = jnp.pad(src, ((0, 0), (0, pad)), constant_values=0).reshape(NT, NB, B)
  dstp = jnp.pad(dst, ((0, 0), (0, pad)), constant_values=N).reshape(NT, NB, B)
  ones_h = jnp.ones((B, CKD), _MSG)
  zeros_h = jnp.zeros((ACC_ROWS, CKD), _MSG)

  degf = _deg_call(dstp, ones_h, zeros_h)
  y = _tc0(x, W0, degf)
  bs = [b0, b1, b2, b3]
  Ws = [W1, W2, W3]
  for l in range(3):
    acc = _agg_call(y, srcp, dstp)
    y = _tcmid(acc, Ws[l], bs[l], degf)
  acc = _agg_call(y, srcp, dstp)
  out = _tcfinal(acc, bs[3], degf, batch, descriptors, Wd, bd, Wlin, blin)
  return out.reshape(-1)


# speedup vs baseline: 2.0161x; 1.0132x over previous
import jax
import jax.numpy as jnp
from jax import lax
from jax.experimental import pallas as pl
from jax.experimental.pallas import tpu as pltpu
from jax.experimental.pallas import tpu_sc as plsc

N = 10000
E = 160000
CH0 = 256
H = 512
G = 64
DESC = 128

CK = 128
NCK = H // CK
NT = 16
NC = 2
NPC = NCK // NC
EPT = E // NT
B = 128
D = 3
NGRP = (EPT + B * D - 1) // (B * D)
NB = NGRP * D
EPP = NB * B
RPT = 632
RPT_LAST = N - (NT - 1) * RPT
ACC_ROWS = NT * RPT

_f32 = jnp.float32
_MSG = jnp.bfloat16


def _copy_rows(s, src_ref, dst_ref):
  r0 = pl.multiple_of(s * RPT, 8)

  @pl.when(s < NT - 1)
  def _():
    pltpu.sync_copy(src_ref.at[pl.ds(r0, RPT)], dst_ref.at[pl.ds(r0, RPT)])

  @pl.when(s == NT - 1)
  def _():
    base = (NT - 1) * RPT
    pltpu.sync_copy(src_ref.at[pl.ds(base, RPT_LAST)],
                    dst_ref.at[pl.ds(base, RPT_LAST)])


def _sc_mesh():
  return plsc.VectorSubcoreMesh(
      core_axis_name="c", subcore_axis_name="s",
      num_cores=NC, num_subcores=NT)


def _agg_body(*refs):
  ys = refs[:NCK]
  srcp, dstp = refs[NCK:NCK + 2]
  os_ = refs[NCK + 2:2 * NCK + 2]
  src_v, dst_v, bufs, gsems, ssems, acc, ysh = refs[2 * NCK + 2:]
  c = lax.axis_index("c")
  s = lax.axis_index("s")
  pltpu.sync_copy(srcp.at[s], src_v)
  pltpu.sync_copy(dstp.at[s], dst_v)

  def process(y_ref, o_ref):
    _copy_rows(s, y_ref, ysh)
    _copy_rows(s, y_ref, acc)
    plsc.subcore_barrier()

    def fire_gathers(base):
      return [pltpu.async_copy(
          ysh.at[src_v.at[base + k]], bufs.at[k], gsems.at[k])
          for k in range(D)]

    def fire_scatters(base, descs):
      for k in range(D):
        descs[k].wait()
        pltpu.async_copy(bufs.at[k], acc.at[dst_v.at[base + k]],
                         ssems.at[k], add=True)

    fire_scatters(0, fire_gathers(0))

    def body(g, carry):
      base = g * D
      descs = []
      for k in range(D):
        pltpu.make_async_copy(
            y_ref.at[pl.ds(0, B)], bufs.at[k], ssems.at[k]).wait()
        descs.append(pltpu.async_copy(
            ysh.at[src_v.at[base + k]], bufs.at[k], gsems.at[k]))
      fire_scatters(base, descs)
      return carry

    lax.fori_loop(1, NGRP, body, 0)
    for k in range(D):
      pltpu.make_async_copy(
          y_ref.at[pl.ds(0, B)], bufs.at[k], ssems.at[k]).wait()
    plsc.subcore_barrier()
    _copy_rows(s, acc, o_ref)
    plsc.subcore_barrier()

  @pl.when(c == 0)
  def _():
    for k in range(NPC):
      process(ys[k], os_[k])

  @pl.when(c == 1)
  def _():
    for k in range(NPC):
      process(ys[NPC + k], os_[NPC + k])


def _agg_call(y_chunks, srcp, dstp):
  fn = pl.kernel(
      _agg_body,
      out_type=[jax.ShapeDtypeStruct((N, CK), _MSG)] * NCK,
      mesh=_sc_mesh(),
      scratch_types=[
          pltpu.VMEM((NB, B), jnp.int32),
          pltpu.VMEM((NB, B), jnp.int32),
          pltpu.VMEM((D, B, CK), _MSG),
          pltpu.SemaphoreType.DMA((D,)),
          pltpu.SemaphoreType.DMA((D,)),
          pltpu.VMEM_SHARED((ACC_ROWS, CK), _MSG),
          pltpu.VMEM_SHARED((ACC_ROWS, CK), _MSG),
      ],
      compiler_params=pltpu.CompilerParams(use_tc_tiling_on_sc=False),
  )
  return fn(*y_chunks, srcp, dstp)


CKD = 32


NB0 = (NB // (2 * D) + 1) * D
NG0 = NB0 // D
NG1 = (NB - NB0) // D


def _deg_body(dstp, ones_h, zeros_h, dega, degb, dst_v, buf, ssems, acc):
  c = lax.axis_index("c")
  s = lax.axis_index("s")
  pltpu.sync_copy(dstp.at[s], dst_v)
  pltpu.sync_copy(ones_h, buf)
  z0 = pl.multiple_of(s * RPT, 8)
  pltpu.sync_copy(zeros_h.at[pl.ds(z0, RPT)], acc.at[pl.ds(z0, RPT)])
  plsc.subcore_barrier()

  def run(base0, ngrp, o_ref):
    for k in range(D):
      pltpu.async_copy(buf, acc.at[dst_v.at[base0 + k]], ssems.at[k],
                       add=True)

    def body(g, carry):
      for k in range(D):
        pltpu.make_async_copy(ones_h, buf, ssems.at[k]).wait()
        pltpu.async_copy(buf, acc.at[dst_v.at[base0 + g * D + k]],
                         ssems.at[k], add=True)
      return carry

    lax.fori_loop(1, ngrp, body, 0)
    for k in range(D):
      pltpu.make_async_copy(ones_h, buf, ssems.at[k]).wait()
    plsc.subcore_barrier()
    _copy_rows(s, acc, o_ref)

  @pl.when(c == 0)
  def _():
    run(0, NG0, dega)

  @pl.when(c == 1)
  def _():
    run(NB0, NG1, degb)


def _deg_call(dstp, ones_h, zeros_h):
  fn = pl.kernel(
      _deg_body,
      out_type=[jax.ShapeDtypeStruct((N, CKD), _MSG)] * 2,
      mesh=_sc_mesh(),
      scratch_types=[
          pltpu.VMEM((NB, B), jnp.int32),
          pltpu.VMEM((B, CKD), _MSG),
          pltpu.SemaphoreType.DMA((D,)),
          pltpu.VMEM_SHARED((ACC_ROWS, CKD), _MSG),
      ],
      compiler_params=pltpu.CompilerParams(use_tc_tiling_on_sc=False),
  )
  return fn(dstp, ones_h, zeros_h)


R0 = 2000


def _tc0_body(x_ref, w_ref, da_ref, db_ref, *y_refs):
  deg = da_ref[:, 0:1].astype(_f32) + db_ref[:, 0:1].astype(_f32)
  dinv = lax.rsqrt(deg + 1.0)
  y = jnp.dot((x_ref[...] * dinv).astype(_MSG), w_ref[...],
              preferred_element_type=_f32)
  for k, yr in enumerate(y_refs):
    yr[...] = y[:, k * CK:(k + 1) * CK].astype(_MSG)


def _tc0(x, W, dega, degb):
  return pl.pallas_call(
      _tc0_body,
      grid=(N // R0,),
      in_specs=[
          pl.BlockSpec((R0, CH0), lambda i: (i, 0)),
          pl.BlockSpec((CH0, H), lambda i: (0, 0)),
          pl.BlockSpec((R0, CKD), lambda i: (i, 0)),
          pl.BlockSpec((R0, CKD), lambda i: (i, 0)),
      ],
      out_specs=[pl.BlockSpec((R0, CK), lambda i: (i, 0))] * NCK,
      out_shape=[jax.ShapeDtypeStruct((N, CK), _MSG)] * NCK,
  )(x, W.astype(_MSG), dega, degb)


def _tcmid_body(*refs):
  a_refs = refs[:NCK]
  w_ref, da_ref, db_ref, b_ref = refs[NCK:NCK + 4]
  y_refs = refs[NCK + 4:]
  deg = da_ref[:, 0:1].astype(_f32) + db_ref[:, 0:1].astype(_f32)
  dinv = lax.rsqrt(deg + 1.0)
  h = jnp.concatenate([a[...].astype(_f32) for a in a_refs], axis=1)
  h = jnp.maximum(h * dinv + b_ref[...], 0.0)
  y = jnp.dot((h * dinv).astype(_MSG), w_ref[...],
              preferred_element_type=_f32)
  for k, yr in enumerate(y_refs):
    yr[...] = y[:, k * CK:(k + 1) * CK].astype(_MSG)


def _tcmid(acc, W, b, dega, degb):
  return pl.pallas_call(
      _tcmid_body,
      grid=(N // R0,),
      in_specs=[pl.BlockSpec((R0, CK), lambda i: (i, 0))] * NCK + [
          pl.BlockSpec((H, H), lambda i: (0, 0)),
          pl.BlockSpec((R0, CKD), lambda i: (i, 0)),
          pl.BlockSpec((R0, CKD), lambda i: (i, 0)),
          pl.BlockSpec((1, H), lambda i: (0, 0)),
      ],
      out_specs=[pl.BlockSpec((R0, CK), lambda i: (i, 0))] * NCK,
      out_shape=[jax.ShapeDtypeStruct((N, CK), _MSG)] * NCK,
  )(*acc, W.astype(_MSG), dega, degb, b.reshape(1, H))


RF = 1000
NGF = N // RF


def _fin_body(*refs):
  a_refs = refs[:NCK]
  (da_ref, db_ref, b_ref, batch_ref, desc_ref, wd_ref, bd_ref, wl_ref,
   bl_ref, out_ref, sums, counts) = refs[NCK:]
  i = pl.program_id(0)

  @pl.when(i == 0)
  def _():
    sums[...] = jnp.zeros_like(sums)
    counts[...] = jnp.zeros_like(counts)

  deg = da_ref[:, 0:1].astype(_f32) + db_ref[:, 0:1].astype(_f32)
  dinv = lax.rsqrt(deg + 1.0)
  h = jnp.concatenate([a[...].astype(_f32) for a in a_refs], axis=1)
  h = jnp.maximum(h * dinv + b_ref[...], 0.0)
  gids = lax.broadcasted_iota(jnp.int32, (RF, G), 1)
  P = (batch_ref[...] == gids).astype(_f32)
  sums[...] += lax.dot_general(P, h, (((0,), (0,)), ((), ())),
                               preferred_element_type=_f32)
  counts[...] += lax.dot_general(P, jnp.ones((RF, 1), _f32),
                                 (((0,), (0,)), ((), ())),
                                 preferred_element_type=_f32)

  @pl.when(i == NGF - 1)
  def _():
    gm = sums[...] / jnp.maximum(counts[...], 1.0)
    de = jnp.maximum(
        jnp.dot(desc_ref[...], wd_ref[...], preferred_element_type=_f32)
        + bd_ref[...], 0.0)
    z = jnp.concatenate([gm, de], axis=1)
    logit = jnp.dot(z, wl_ref[...], preferred_element_type=_f32) + bl_ref[...]
    out_ref[...] = jax.nn.sigmoid(logit)


def _tcfinal(acc, b, dega, degb, batch, descriptors, Wd, bd, Wlin, blin):
  return pl.pallas_call(
      _fin_body,
      grid=(NGF,),
      in_specs=[pl.BlockSpec((RF, CK), lambda i: (i, 0))] * NCK + [
          pl.BlockSpec((RF, CKD), lambda i: (i, 0)),
          pl.BlockSpec((RF, CKD), lambda i: (i, 0)),
          pl.BlockSpec((1, H), lambda i: (0, 0)),
          pl.BlockSpec((RF, 1), lambda i: (i, 0)),
          pl.BlockSpec((G, DESC), lambda i: (0, 0)),
          pl.BlockSpec((DESC, H), lambda i: (0, 0)),
          pl.BlockSpec((1, H), lambda i: (0, 0)),
          pl.BlockSpec((2 * H, 1), lambda i: (0, 0)),
          pl.BlockSpec((1, 1), lambda i: (0, 0)),
      ],
      out_specs=pl.BlockSpec((G, 1), lambda i: (0, 0)),
      out_shape=jax.ShapeDtypeStruct((G, 1), _f32),
      scratch_shapes=[
          pltpu.VMEM((G, H), _f32),
          pltpu.VMEM((G, 1), _f32),
      ],
  )(*acc, dega, degb, b.reshape(1, H), batch.reshape(N, 1), descriptors,
    Wd, bd.reshape(1, H), Wlin, blin.reshape(1, 1))


def kernel(x, edge_index, batch, descriptors,
           W0, b0, W1, b1, W2, b2, W3, b3, Wd, bd, Wlin, blin):
  src = edge_index[0].reshape(NT, EPT)
  dst = edge_index[1].reshape(NT, EPT)
  pad = EPP - EPT
  srcp = jnp.pad(src, ((0, 0), (0, pad)), constant_values=0).reshape(NT, NB, B)
  dstp = jnp.pad(dst, ((0, 0), (0, pad)), constant_values=N).reshape(NT, NB, B)
  ones_h = jnp.ones((B, CKD), _MSG)
  zeros_h = jnp.zeros((ACC_ROWS, CKD), _MSG)

  dega, degb = _deg_call(dstp, ones_h, zeros_h)
  y = _tc0(x, W0, dega, degb)
  bs = [b0, b1, b2, b3]
  Ws = [W1, W2, W3]
  for l in range(3):
    acc = _agg_call(y, srcp, dstp)
    y = _tcmid(acc, Ws[l], bs[l], dega, degb)
  acc = _agg_call(y, srcp, dstp)
  out = _tcfinal(acc, bs[3], dega, degb, batch, descriptors,
                 Wd, bd, Wlin, blin)
  return out.reshape(-1)
